# baseline (speedup 1.0000x reference)
import functools
import numpy as np
import jax
import jax.numpy as jnp
from jax import lax
from jax.experimental import pallas as pl
from jax.experimental.pallas import tpu as pltpu
from jax.experimental.pallas import tpu_sc as plsc

_VOCAB = 1000000
_EMBED = 64
_SEQ = 200
_BATCH = 4096

_NC = 2
_NS = 16
_NW = _NC * _NS
_ROWS = _BATCH * _SEQ
_ROWS_W = _ROWS // _NW
_IDXW = 128
_K = 4
_CHUNK = _K * _IDXW
_NITER = _ROWS_W // _CHUNK
_SCALE = float(np.sqrt(_EMBED))


def _positional_encoding_flat():
    depth_h = _EMBED / 2
    positions = np.arange(_SEQ)[:, np.newaxis]
    depths = np.arange(depth_h)[np.newaxis, :] / depth_h
    angle_rates = 1 / 10000 ** depths
    angle_rads = positions * angle_rates
    pos = np.concatenate([np.sin(angle_rads), np.cos(angle_rads)], axis=-1)
    return pos.reshape(-1).astype(np.float32)


_POS_FLAT = _positional_encoding_flat()


def _body(x_hbm, pos_hbm, table_hbm, out_hbm, idx_v, pos_v, rows_v, sem):
    c = lax.axis_index("c")
    s = lax.axis_index("s")
    wid = s * _NC + c
    pltpu.sync_copy(x_hbm.at[wid], idx_v)
    pltpu.sync_copy(pos_hbm, pos_v)
    row0 = wid * _ROWS_W

    def step(g, carry):
        copies = [
            pltpu.async_copy(
                table_hbm.at[idx_v.at[g * _K + k]],
                rows_v.at[pl.ds(k * _IDXW, _IDXW)],
                sem,
            )
            for k in range(_K)
        ]
        for cp in copies:
            cp.wait()

        def row(i, carry2):
            p = lax.rem(g * _CHUNK + i, _SEQ) * _EMBED
            for j in range(_EMBED // 16):
                v = rows_v[i, pl.ds(j * 16, 16)]
                pv = pos_v[pl.ds(p + j * 16, 16)]
                rows_v[i, pl.ds(j * 16, 16)] = v * _SCALE + pv
            return carry2

        lax.fori_loop(0, _CHUNK, row, 0)
        pltpu.sync_copy(rows_v, out_hbm.at[pl.ds(row0 + g * _CHUNK, _CHUNK)])
        return carry

    lax.fori_loop(0, _NITER, step, 0)


@functools.partial(jax.jit, donate_argnums=())
def _run(x_r, table):
    pos = jnp.asarray(_POS_FLAT)
    fn = pl.kernel(
        _body,
        out_type=jax.ShapeDtypeStruct((_ROWS, _EMBED), jnp.float32),
        mesh=plsc.VectorSubcoreMesh(
            core_axis_name="c", subcore_axis_name="s",
            num_cores=_NC, num_subcores=_NS,
        ),
        scratch_types=[
            pltpu.VMEM((_ROWS_W // _IDXW, _IDXW), jnp.int32),
            pltpu.VMEM((_SEQ * _EMBED,), jnp.float32),
            pltpu.VMEM((_CHUNK, _EMBED), jnp.float32),
            pltpu.SemaphoreType.DMA,
        ],
        compiler_params=pltpu.CompilerParams(use_tc_tiling_on_sc=False),
    )
    return fn(x_r, pos, table)


def kernel(x, table):
    x_r = jnp.reshape(x.astype(jnp.int32), (_NW, _ROWS_W // _IDXW, _IDXW))
    out = _run(x_r, table)
    return jnp.reshape(out, (_BATCH, _SEQ, _EMBED))


# speedup vs baseline: 1.0240x; 1.0240x over previous
import functools
import numpy as np
import jax
import jax.numpy as jnp
from jax import lax
from jax.experimental import pallas as pl
from jax.experimental.pallas import tpu as pltpu
from jax.experimental.pallas import tpu_sc as plsc

_VOCAB = 1000000
_EMBED = 64
_SEQ = 200
_BATCH = 4096

_NC = 2
_NS = 16
_NW = _NC * _NS
_BW = _BATCH // _NW
_ROWS = _BATCH * _SEQ
_NBUF = 2
_SCALE = float(np.sqrt(_EMBED))
_NVJ = _EMBED // 16


def _positional_encoding():
    depth_h = _EMBED / 2
    positions = np.arange(_SEQ)[:, np.newaxis]
    depths = np.arange(depth_h)[np.newaxis, :] / depth_h
    angle_rates = 1 / 10000 ** depths
    angle_rads = positions * angle_rates
    pos = np.concatenate([np.sin(angle_rads), np.cos(angle_rads)], axis=-1)
    return pos.astype(np.float32)


_POS = _positional_encoding()


def _body(x_hbm, pos_hbm, table_hbm, out_hbm,
          idx_v, pos_v, grows, souts, oidx_v, obase_v, gsem, ssem):
    c = lax.axis_index("c")
    s = lax.axis_index("s")
    wid = s * _NC + c
    pltpu.sync_copy(x_hbm.at[wid], idx_v)
    pltpu.sync_copy(pos_hbm, pos_v)

    iota = lax.iota(jnp.int32, 16)
    for k in range(_BW // 16):
        obase_v[pl.ds(k * 16, 16)] = (iota + (wid * _BW + k * 16)) * _SEQ

    def gather(p, slot):
        pltpu.async_copy(table_hbm.at[idx_v.at[p]], grows.at[slot],
                         gsem.at[slot])

    for p0 in range(_NBUF):
        gather(p0, p0)

    def step(p, carry):
        slot = lax.rem(p, _NBUF)

        @pl.when(p >= _NBUF)
        def _():
            pltpu.make_async_copy(souts.at[slot],
                                  out_hbm.at[oidx_v.at[slot]],
                                  ssem.at[slot]).wait()

        pltpu.make_async_copy(table_hbm.at[idx_v.at[p]], grows.at[slot],
                              gsem.at[slot]).wait()

        pv = [pos_v[p, pl.ds(j * 16, 16)] for j in range(_NVJ)]

        def row(b, carry2):
            for j in range(_NVJ):
                souts[slot, b, pl.ds(j * 16, 16)] = (
                    grows[slot, b, pl.ds(j * 16, 16)] * _SCALE + pv[j])
            return carry2

        lax.fori_loop(0, _BW, row, 0, unroll=4)

        for k in range(_BW // 16):
            oidx_v[slot, pl.ds(k * 16, 16)] = obase_v[pl.ds(k * 16, 16)] + p

        pltpu.async_copy(souts.at[slot], out_hbm.at[oidx_v.at[slot]],
                         ssem.at[slot])

        @pl.when(p + _NBUF < _SEQ)
        def _():
            gather(p + _NBUF, slot)

        return carry

    lax.fori_loop(0, _SEQ, step, 0)

    for r in range(_NBUF):
        slot = (_SEQ - _NBUF + r) % _NBUF
        pltpu.make_async_copy(souts.at[slot], out_hbm.at[oidx_v.at[slot]],
                              ssem.at[slot]).wait()


@jax.jit
def _run(x_r, table):
    pos = jnp.asarray(_POS)
    fn = pl.kernel(
        _body,
        out_type=jax.ShapeDtypeStruct((_ROWS, _EMBED), jnp.float32),
        mesh=plsc.VectorSubcoreMesh(
            core_axis_name="c", subcore_axis_name="s",
            num_cores=_NC, num_subcores=_NS,
        ),
        scratch_types=[
            pltpu.VMEM((_SEQ, _BW), jnp.int32),
            pltpu.VMEM((_SEQ, _EMBED), jnp.float32),
            pltpu.VMEM((_NBUF, _BW, _EMBED), jnp.float32),
            pltpu.VMEM((_NBUF, _BW, _EMBED), jnp.float32),
            pltpu.VMEM((_NBUF, _BW), jnp.int32),
            pltpu.VMEM((_BW,), jnp.int32),
            pltpu.SemaphoreType.DMA((_NBUF,)),
            pltpu.SemaphoreType.DMA((_NBUF,)),
        ],
        compiler_params=pltpu.CompilerParams(use_tc_tiling_on_sc=False),
    )
    return fn(x_r, pos, table)


def kernel(x, table):
    x_r = jnp.swapaxes(
        jnp.reshape(x.astype(jnp.int32), (_NW, _BW, _SEQ)), 1, 2)
    out = _run(x_r, table)
    return jnp.reshape(out, (_BATCH, _SEQ, _EMBED))
